# Initial kernel scaffold; baseline (speedup 1.0000x reference)
import functools

import jax
import jax.numpy as jnp
from jax import lax
from jax.experimental import pallas as pl
from jax.experimental.pallas import tpu as pltpu
from jax.experimental.pallas import tpu_sc as plsc

NC = 2
NS = 16
L = 16


def _make_spmm(n, e_per_branch, d, offset_cols, chunk=80):
  epw = e_per_branch // NS
  assert epw % chunk == 0
  n_chunks = epw // chunk
  rows_per_tile = n // NS
  zr = 125
  assert rows_per_tile % zr == 0
  n_stage = rows_per_tile // zr

  def body(table_ref, rows_ref, cols_ref, vals_ref, out_ref,
           idx_r, idx_c, vls, gbuf, zbuf, acc, sem):
    c = lax.axis_index("c")
    s = lax.axis_index("s")

    def zero_row(i, carry):
      for q in range(d // L):
        zbuf[i, pl.ds(q * L, L)] = jnp.zeros((L,), jnp.float32)
      return carry
    lax.fori_loop(0, zr, zero_row, 0)
    for k in range(n_stage):
      pltpu.sync_copy(zbuf, acc.at[pl.ds(s * rows_per_tile + k * zr, zr)])
    plsc.subcore_barrier()

    base0 = c * e_per_branch + s * epw

    def chunk_body(k, carry):
      b = base0 + k * chunk
      pltpu.sync_copy(rows_ref.at[pl.ds(b, chunk)], idx_r)
      pltpu.sync_copy(cols_ref.at[pl.ds(b, chunk)], idx_c)
      pltpu.sync_copy(vals_ref.at[pl.ds(b, chunk)], vls)
      if offset_cols:
        off = c * n
        for j in range(chunk // L):
          idx_c[pl.ds(j * L, L)] = idx_c[pl.ds(j * L, L)] + off
      pltpu.async_copy(table_ref.at[idx_c], gbuf, sem).wait()

      def edge_body(ei, carry2):
        v = vls[ei]
        for q in range(d // L):
          gbuf[ei, pl.ds(q * L, L)] = gbuf[ei, pl.ds(q * L, L)] * v
        return carry2
      lax.fori_loop(0, chunk, edge_body, 0)

      pltpu.sync_copy(gbuf, acc.at[idx_r], add=True)
      return carry
    lax.fori_loop(0, n_chunks, chunk_body, 0)

    plsc.subcore_barrier()
    for k in range(n_stage):
      r0 = s * rows_per_tile + k * zr
      pltpu.sync_copy(acc.at[pl.ds(r0, zr)], zbuf)
      pltpu.sync_copy(zbuf, out_ref.at[c, pl.ds(r0, zr)])

  return pl.kernel(
      body,
      out_type=jax.ShapeDtypeStruct((NC, n, d), jnp.float32),
      mesh=plsc.VectorSubcoreMesh(
          core_axis_name="c", subcore_axis_name="s",
          num_cores=NC, num_subcores=NS),
      scratch_types=[
          pltpu.VMEM((chunk,), jnp.int32),
          pltpu.VMEM((chunk,), jnp.int32),
          pltpu.VMEM((chunk,), jnp.float32),
          pltpu.VMEM((chunk, d), jnp.float32),
          pltpu.VMEM((zr, d), jnp.float32),
          pltpu.VMEM_SHARED((n, d), jnp.float32),
          pltpu.SemaphoreType.DMA,
      ],
  )


def _gcn_dense(s1, w1s, b1s, w2s, bm=1000):
  nb, n, f = s1.shape
  h2 = w1s.shape[2]
  h = w2s.shape[2]

  def body(s1_ref, w1_ref, b1_ref, w2_ref, y_ref):
    hh = jnp.maximum(
        jnp.dot(s1_ref[0], w1_ref[0], preferred_element_type=jnp.float32)
        + b1_ref[0], 0.0)
    y_ref[0] = jnp.dot(hh, w2_ref[0], preferred_element_type=jnp.float32)

  return pl.pallas_call(
      body,
      grid=(nb, n // bm),
      in_specs=[
          pl.BlockSpec((1, bm, f), lambda b, i: (b, i, 0)),
          pl.BlockSpec((1, f, h2), lambda b, i: (b, 0, 0)),
          pl.BlockSpec((1, h2), lambda b, i: (b, 0)),
          pl.BlockSpec((1, h2, h), lambda b, i: (b, 0, 0)),
      ],
      out_specs=pl.BlockSpec((1, bm, h), lambda b, i: (b, i, 0)),
      out_shape=jax.ShapeDtypeStruct((nb, n, h), jnp.float32),
  )(s1, w1s, b1s, w2s)


def _lstm_diff(g, b2s, wih_t, whh_t, b_ih, b_hh, h_prev, c_prev, bm=1000):
  _, n, hdim = g.shape

  def body(g_ref, b2_ref, wih_ref, whh_ref, bih_ref, bhh_ref, hp_ref, cp_ref,
           d_ref):
    bias = bih_ref[...] + bhh_ref[...]

    def branch(b):
      gb = g_ref[b] + b2_ref[b]
      gates = (jnp.dot(gb, wih_ref[...], preferred_element_type=jnp.float32)
               + jnp.dot(hp_ref[b], whh_ref[...],
                         preferred_element_type=jnp.float32)
               + bias)
      i_g = jax.nn.sigmoid(gates[:, :hdim])
      f_g = jax.nn.sigmoid(gates[:, hdim:2 * hdim])
      g_g = jnp.tanh(gates[:, 2 * hdim:3 * hdim])
      o_g = jax.nn.sigmoid(gates[:, 3 * hdim:])
      c_new = f_g * cp_ref[b] + i_g * g_g
      return o_g * jnp.tanh(c_new)

    d_ref[...] = branch(0) - branch(1)

  return pl.pallas_call(
      body,
      grid=(n // bm,),
      in_specs=[
          pl.BlockSpec((2, bm, hdim), lambda i: (0, i, 0)),
          pl.BlockSpec((2, hdim), lambda i: (0, 0)),
          pl.BlockSpec((hdim, 4 * hdim), lambda i: (0, 0)),
          pl.BlockSpec((hdim, 4 * hdim), lambda i: (0, 0)),
          pl.BlockSpec((4 * hdim,), lambda i: (0,)),
          pl.BlockSpec((4 * hdim,), lambda i: (0,)),
          pl.BlockSpec((2, bm, hdim), lambda i: (0, i, 0)),
          pl.BlockSpec((2, bm, hdim), lambda i: (0, i, 0)),
      ],
      out_specs=pl.BlockSpec((bm, hdim), lambda i: (i, 0)),
      out_shape=jax.ShapeDtypeStruct((n, hdim), jnp.float32),
  )(g, b2s, wih_t, whh_t, b_ih, b_hh, h_prev, c_prev)


def _head(adj, d, feat, hid, wf, bf, w1, b1, w2, b2, bm=250):
  n = adj.shape[0]
  hdim = d.shape[1]
  f = feat.shape[1]
  o = w2.shape[1]

  def body(adj_ref, d_ref, feat_ref, hid_ref, wf_ref, bf_ref, w1_ref, b1_ref,
           w2_ref, b2_ref, ne_ref, y_ref):
    acc = jnp.dot(adj_ref[...], d_ref[...], preferred_element_type=jnp.float32)
    ne = (hid_ref[...] + bf_ref[...]
          + jnp.dot(feat_ref[...], wf_ref[...],
                    preferred_element_type=jnp.float32)
          + acc)
    ne_ref[...] = ne
    t = jnp.maximum(
        jnp.dot(ne, w1_ref[...], preferred_element_type=jnp.float32)
        + b1_ref[...], 0.0)
    y_ref[...] = (jnp.dot(t, w2_ref[...], preferred_element_type=jnp.float32)
                  + b2_ref[...])

  return pl.pallas_call(
      body,
      grid=(n // bm,),
      in_specs=[
          pl.BlockSpec((bm, n), lambda i: (i, 0)),
          pl.BlockSpec((n, hdim), lambda i: (0, 0)),
          pl.BlockSpec((bm, f), lambda i: (i, 0)),
          pl.BlockSpec((bm, hdim), lambda i: (i, 0)),
          pl.BlockSpec((f, hdim), lambda i: (0, 0)),
          pl.BlockSpec((hdim,), lambda i: (0,)),
          pl.BlockSpec((hdim, hdim), lambda i: (0, 0)),
          pl.BlockSpec((hdim,), lambda i: (0,)),
          pl.BlockSpec((hdim, o), lambda i: (0, 0)),
          pl.BlockSpec((o,), lambda i: (0,)),
      ],
      out_specs=[
          pl.BlockSpec((bm, hdim), lambda i: (i, 0)),
          pl.BlockSpec((bm, o), lambda i: (i, 0)),
      ],
      out_shape=[
          jax.ShapeDtypeStruct((n, hdim), jnp.float32),
          jax.ShapeDtypeStruct((n, o), jnp.float32),
      ],
  )(adj, d, feat, hid, wf, bf, w1, b1, w2, b2)


def kernel(features, edge_index_v, edge_vals_v, edge_index_p, edge_vals_p,
           adj_d, last_hidden_v, last_c_v, last_hidden_p, last_c_p,
           hidden_embedding, params):
  p = params
  n, f = features.shape
  e = edge_vals_v.shape[0]
  fp = 32

  feat32 = jnp.pad(features, ((0, 0), (0, fp - f)))
  rows = jnp.concatenate([edge_index_v[0], edge_index_p[0]])
  cols = jnp.concatenate([edge_index_v[1], edge_index_p[1]])
  vals = jnp.concatenate([edge_vals_v, edge_vals_p])

  w1s = jnp.stack([
      jnp.pad(p['gcn_v_W1'], ((0, fp - f), (0, 0))),
      jnp.pad(p['gcn_p_W1'], ((0, fp - f), (0, 0))),
  ])
  b1s = jnp.stack([p['gcn_v_b1'], p['gcn_p_b1']])
  w2s = jnp.stack([p['gcn_v_W2'], p['gcn_p_W2']])
  b2s = jnp.stack([p['gcn_v_b2'], p['gcn_p_b2']])

  s1 = _make_spmm(n, e, fp, offset_cols=False)(feat32, rows, cols, vals)
  y = _gcn_dense(s1, w1s, b1s, w2s)
  g = _make_spmm(n, e, y.shape[2], offset_cols=True)(
      y.reshape(2 * n, y.shape[2]), rows, cols, vals)

  h_prev = jnp.stack([last_hidden_v, last_hidden_p])
  c_prev = jnp.stack([last_c_v, last_c_p])
  d = _lstm_diff(g, b2s, p['lstm_W_ih'].T, p['lstm_W_hh'].T,
                 p['lstm_b_ih'], p['lstm_b_hh'], h_prev, c_prev)

  wf = jnp.pad(p['lin_Wf'], ((0, fp - f), (0, 0)))
  node_embedding, y_pred = _head(
      adj_d, d, feat32, hidden_embedding, wf, p['lin_bf'],
      p['mlp_W1'], p['mlp_b1'], p['mlp_W2'], p['mlp_b2'])
  return node_embedding, y_pred


# speedup vs baseline: 3.5715x; 3.5715x over previous
import functools

import jax
import jax.numpy as jnp
from jax import lax
from jax.experimental import pallas as pl
from jax.experimental.pallas import tpu as pltpu
from jax.experimental.pallas import tpu_sc as plsc

NC = 2
NS = 16
L = 16


def _make_spmm(n, e_per_branch, d, offset_cols, chunk=80):
  epw = e_per_branch // NS
  assert epw % chunk == 0
  n_chunks = epw // chunk
  rows_per_tile = -(-n // (NS * 8)) * 8
  n_pad = rows_per_tile * NS
  assert rows_per_tile % 4 == 0
  zr = rows_per_tile // 4

  def body(table_ref, rows_ref, cols_ref, vals_ref, out_ref,
           idx_r, idx_c, vls, gbuf, zbuf, acc, sem):
    c = lax.axis_index("c")
    s = lax.axis_index("s")

    def zero_row(i, carry):
      for q in range(d // L):
        zbuf[i, pl.ds(q * L, L)] = jnp.zeros((L,), jnp.float32)
      return carry
    lax.fori_loop(0, zr, zero_row, 0)
    for k in range(4):
      pltpu.sync_copy(zbuf, acc.at[pl.ds(s * rows_per_tile + k * zr, zr)])
    plsc.subcore_barrier()

    base0 = c * e_per_branch + s * epw

    def chunk_body(k, carry):
      b = base0 + k * chunk
      pltpu.sync_copy(rows_ref.at[pl.ds(b, chunk)], idx_r)
      pltpu.sync_copy(cols_ref.at[pl.ds(b, chunk)], idx_c)
      pltpu.sync_copy(vals_ref.at[pl.ds(b, chunk)], vls)
      if offset_cols:
        off = c * n
        for j in range(chunk // L):
          idx_c[pl.ds(j * L, L)] = idx_c[pl.ds(j * L, L)] + off
      pltpu.async_copy(table_ref.at[idx_c], gbuf, sem).wait()

      def grp_body(jj, carry2):
        vv = vls[pl.ds(jj * L, L)]
        for i in range(L):
          v = vv[i]
          ei = jj * L + i
          for q in range(d // L):
            gbuf[ei, pl.ds(q * L, L)] = gbuf[ei, pl.ds(q * L, L)] * v
        return carry2
      lax.fori_loop(0, chunk // L, grp_body, 0)

      pltpu.sync_copy(gbuf, acc.at[idx_r], add=True)
      return carry
    lax.fori_loop(0, n_chunks, chunk_body, 0)

    plsc.subcore_barrier()
    for k in range(4):
      r0 = s * rows_per_tile + k * zr
      pltpu.sync_copy(acc.at[pl.ds(r0, zr)], zbuf)
      pltpu.sync_copy(zbuf, out_ref.at[c, pl.ds(r0, zr)])

  return pl.kernel(
      body,
      out_type=jax.ShapeDtypeStruct((NC, n_pad, d), jnp.float32),
      mesh=plsc.VectorSubcoreMesh(
          core_axis_name="c", subcore_axis_name="s",
          num_cores=NC, num_subcores=NS),
      scratch_types=[
          pltpu.VMEM((chunk,), jnp.int32),
          pltpu.VMEM((chunk,), jnp.int32),
          pltpu.VMEM((chunk,), jnp.float32),
          pltpu.VMEM((chunk, d), jnp.float32),
          pltpu.VMEM((zr, d), jnp.float32),
          pltpu.VMEM_SHARED((n_pad, d), jnp.float32),
          pltpu.SemaphoreType.DMA,
      ],
      compiler_params=pltpu.CompilerParams(use_tc_tiling_on_sc=False),
  )


def _gcn_dense(s1, w1s, b1s, w2s, bm=1000):
  nb, n, f = s1.shape
  h2 = w1s.shape[2]
  h = w2s.shape[2]

  def body(s1_ref, w1_ref, b1_ref, w2_ref, y_ref):
    hh = jnp.maximum(
        jnp.dot(s1_ref[0], w1_ref[0], preferred_element_type=jnp.float32)
        + b1_ref[0, 0], 0.0)
    y_ref[0] = jnp.dot(hh, w2_ref[0], preferred_element_type=jnp.float32)

  return pl.pallas_call(
      body,
      grid=(nb, n // bm),
      in_specs=[
          pl.BlockSpec((1, bm, f), lambda b, i: (b, i, 0)),
          pl.BlockSpec((1, f, h2), lambda b, i: (b, 0, 0)),
          pl.BlockSpec((1, 1, h2), lambda b, i: (b, 0, 0)),
          pl.BlockSpec((1, h2, h), lambda b, i: (b, 0, 0)),
      ],
      out_specs=pl.BlockSpec((1, bm, h), lambda b, i: (b, i, 0)),
      out_shape=jax.ShapeDtypeStruct((nb, n, h), jnp.float32),
  )(s1, w1s, b1s.reshape(nb, 1, h2), w2s)


def _lstm_diff(g, b2s, wih_t, whh_t, b_ih, b_hh, h_prev, c_prev, bm=1000):
  _, n, hdim = g.shape

  def body(g_ref, b2_ref, wih_ref, whh_ref, bih_ref, bhh_ref, hp_ref, cp_ref,
           d_ref):
    bias = bih_ref[...] + bhh_ref[...]

    def branch(b):
      gb = g_ref[b] + b2_ref[b, 0]
      gates = (jnp.dot(gb, wih_ref[...], preferred_element_type=jnp.float32)
               + jnp.dot(hp_ref[b], whh_ref[...],
                         preferred_element_type=jnp.float32)
               + bias)
      i_g = jax.nn.sigmoid(gates[:, :hdim])
      f_g = jax.nn.sigmoid(gates[:, hdim:2 * hdim])
      g_g = jnp.tanh(gates[:, 2 * hdim:3 * hdim])
      o_g = jax.nn.sigmoid(gates[:, 3 * hdim:])
      c_new = f_g * cp_ref[b] + i_g * g_g
      return o_g * jnp.tanh(c_new)

    d_ref[...] = branch(0) - branch(1)

  return pl.pallas_call(
      body,
      grid=(n // bm,),
      in_specs=[
          pl.BlockSpec((2, bm, hdim), lambda i: (0, i, 0)),
          pl.BlockSpec((2, 1, hdim), lambda i: (0, 0, 0)),
          pl.BlockSpec((hdim, 4 * hdim), lambda i: (0, 0)),
          pl.BlockSpec((hdim, 4 * hdim), lambda i: (0, 0)),
          pl.BlockSpec((4 * hdim,), lambda i: (0,)),
          pl.BlockSpec((4 * hdim,), lambda i: (0,)),
          pl.BlockSpec((2, bm, hdim), lambda i: (0, i, 0)),
          pl.BlockSpec((2, bm, hdim), lambda i: (0, i, 0)),
      ],
      out_specs=pl.BlockSpec((bm, hdim), lambda i: (i, 0)),
      out_shape=jax.ShapeDtypeStruct((n, hdim), jnp.float32),
  )(g, b2s.reshape(2, 1, hdim), wih_t, whh_t, b_ih, b_hh, h_prev, c_prev)


def _head(adj, d, feat, hid, wf, bf, w1, b1, w2, b2, bm=400):
  n = adj.shape[0]
  hdim = d.shape[1]
  f = feat.shape[1]
  o = w2.shape[1]

  def body(adj_ref, d_ref, feat_ref, hid_ref, wf_ref, bf_ref, w1_ref, b1_ref,
           w2_ref, b2_ref, ne_ref, y_ref):
    acc = jnp.dot(adj_ref[...], d_ref[...], preferred_element_type=jnp.float32)
    ne = (hid_ref[...] + bf_ref[...]
          + jnp.dot(feat_ref[...], wf_ref[...],
                    preferred_element_type=jnp.float32)
          + acc)
    ne_ref[...] = ne
    t = jnp.maximum(
        jnp.dot(ne, w1_ref[...], preferred_element_type=jnp.float32)
        + b1_ref[...], 0.0)
    y_ref[...] = (jnp.dot(t, w2_ref[...], preferred_element_type=jnp.float32)
                  + b2_ref[...])

  return pl.pallas_call(
      body,
      grid=(n // bm,),
      in_specs=[
          pl.BlockSpec((bm, n), lambda i: (i, 0)),
          pl.BlockSpec((n, hdim), lambda i: (0, 0)),
          pl.BlockSpec((bm, f), lambda i: (i, 0)),
          pl.BlockSpec((bm, hdim), lambda i: (i, 0)),
          pl.BlockSpec((f, hdim), lambda i: (0, 0)),
          pl.BlockSpec((hdim,), lambda i: (0,)),
          pl.BlockSpec((hdim, hdim), lambda i: (0, 0)),
          pl.BlockSpec((hdim,), lambda i: (0,)),
          pl.BlockSpec((hdim, o), lambda i: (0, 0)),
          pl.BlockSpec((o,), lambda i: (0,)),
      ],
      out_specs=[
          pl.BlockSpec((bm, hdim), lambda i: (i, 0)),
          pl.BlockSpec((bm, o), lambda i: (i, 0)),
      ],
      out_shape=[
          jax.ShapeDtypeStruct((n, hdim), jnp.float32),
          jax.ShapeDtypeStruct((n, o), jnp.float32),
      ],
  )(adj, d, feat, hid, wf, bf, w1, b1, w2, b2)


def kernel(features, edge_index_v, edge_vals_v, edge_index_p, edge_vals_p,
           adj_d, last_hidden_v, last_c_v, last_hidden_p, last_c_p,
           hidden_embedding, params):
  p = params
  n, f = features.shape
  e = edge_vals_v.shape[0]
  fp = 32

  feat32 = jnp.pad(features, ((0, 0), (0, fp - f)))
  rows = jnp.concatenate([edge_index_v[0], edge_index_p[0]])
  cols = jnp.concatenate([edge_index_v[1], edge_index_p[1]])
  vals = jnp.concatenate([edge_vals_v, edge_vals_p])

  w1s = jnp.stack([
      jnp.pad(p['gcn_v_W1'], ((0, fp - f), (0, 0))),
      jnp.pad(p['gcn_p_W1'], ((0, fp - f), (0, 0))),
  ])
  b1s = jnp.stack([p['gcn_v_b1'], p['gcn_p_b1']])
  w2s = jnp.stack([p['gcn_v_W2'], p['gcn_p_W2']])
  b2s = jnp.stack([p['gcn_v_b2'], p['gcn_p_b2']])

  s1 = _make_spmm(n, e, fp, offset_cols=False)(feat32, rows, cols, vals)
  s1 = s1[:, :n, :]
  y = _gcn_dense(s1, w1s, b1s, w2s)
  g = _make_spmm(n, e, y.shape[2], offset_cols=True)(
      y.reshape(2 * n, y.shape[2]), rows, cols, vals)[:, :n, :]

  h_prev = jnp.stack([last_hidden_v, last_hidden_p])
  c_prev = jnp.stack([last_c_v, last_c_p])
  d = _lstm_diff(g, b2s, p['lstm_W_ih'].T, p['lstm_W_hh'].T,
                 p['lstm_b_ih'], p['lstm_b_hh'], h_prev, c_prev)

  wf = jnp.pad(p['lin_Wf'], ((0, fp - f), (0, 0)))
  node_embedding, y_pred = _head(
      adj_d, d, feat32, hidden_embedding, wf, p['lin_bf'],
      p['mlp_W1'], p['mlp_b1'], p['mlp_W2'], p['mlp_b2'])
  return node_embedding, y_pred


# speedup vs baseline: 7.0173x; 1.9648x over previous
import functools

import jax
import jax.numpy as jnp
from jax import lax
from jax.experimental import pallas as pl
from jax.experimental.pallas import tpu as pltpu
from jax.experimental.pallas import tpu_sc as plsc

NC = 2
NS = 16
L = 16


CHUNK = 128


def _make_spmm(n, chunks_per_tile, d, offset_cols):
  rows_per_tile = -(-n // (NS * 8)) * 8
  n_pad = rows_per_tile * NS
  zr = rows_per_tile // 8
  t = chunks_per_tile

  def body(table_ref, packed_ref, out_ref,
           pbuf0, pbuf1, gbuf0, gbuf1, zbuf, acc,
           sem_i0, sem_i1, sem_g0, sem_g1):
    c = lax.axis_index("c")
    s = lax.axis_index("s")
    pbufs, gbufs = (pbuf0, pbuf1), (gbuf0, gbuf1)
    sem_is, sem_gs = (sem_i0, sem_i1), (sem_g0, sem_g1)

    def zero_row(i, carry):
      for q in range(d // L):
        zbuf[i, pl.ds(q * L, L)] = jnp.zeros((L,), jnp.float32)
      return carry
    lax.fori_loop(0, zr, zero_row, 0)
    for k in range(8):
      pltpu.sync_copy(zbuf, acc.at[pl.ds(s * rows_per_tile + k * zr, zr)])
    plsc.subcore_barrier()

    base = (c * NS + s) * t
    col_off = c * n if offset_cols else 0

    def fix_cols(pb):
      if offset_cols:
        for j in range(CHUNK // L):
          pb[1, pl.ds(j * L, L)] = pb[1, pl.ds(j * L, L)] + col_off

    def scale(pb, gb):
      def grp_body(jj, carry2):
        vv = plsc.bitcast(pb[2, pl.ds(jj * L, L)], jnp.float32)
        for i in range(L):
          v = vv[i]
          ei = jj * L + i
          for q in range(d // L):
            gb[ei, pl.ds(q * L, L)] = gb[ei, pl.ds(q * L, L)] * v
        return carry2
      lax.fori_loop(0, CHUNK // L, grp_body, 0)

    pltpu.sync_copy(packed_ref.at[base], pbuf0)
    fix_cols(pbuf0)
    pltpu.async_copy(table_ref.at[pbuf0.at[1]], gbuf0, sem_g0)
    if t > 1:
      pltpu.async_copy(packed_ref.at[base + 1], pbuf1, sem_i1)

    def chunk_body(k, carry):
      b = lax.rem(k, 2)

      @pl.when(k + 1 < t)
      def _issue_next():
        for bb in range(2):
          @pl.when(b == 1 - bb)
          def _():
            pltpu.make_async_copy(
                packed_ref.at[base], pbufs[bb], sem_is[bb]).wait()
            fix_cols(pbufs[bb])
            pltpu.async_copy(
                table_ref.at[pbufs[bb].at[1]], gbufs[bb], sem_gs[bb])

      for bb in range(2):
        @pl.when(b == bb)
        def _():
          pltpu.make_async_copy(
              table_ref.at[pbufs[bb].at[1]], gbufs[bb], sem_gs[bb]).wait()
          scale(pbufs[bb], gbufs[bb])
          pltpu.sync_copy(gbufs[bb], acc.at[pbufs[bb].at[0]], add=True)

          @pl.when(k + 2 < t)
          def _prefetch():
            pltpu.async_copy(packed_ref.at[base + k + 2], pbufs[bb],
                             sem_is[bb])
      return carry
    lax.fori_loop(0, t, chunk_body, 0)

    plsc.subcore_barrier()
    for k in range(8):
      r0 = s * rows_per_tile + k * zr
      pltpu.sync_copy(acc.at[pl.ds(r0, zr)], zbuf)
      pltpu.sync_copy(zbuf, out_ref.at[c, pl.ds(r0, zr)])

  return pl.kernel(
      body,
      out_type=jax.ShapeDtypeStruct((NC, n_pad, d), jnp.float32),
      mesh=plsc.VectorSubcoreMesh(
          core_axis_name="c", subcore_axis_name="s",
          num_cores=NC, num_subcores=NS),
      scratch_types=[
          pltpu.VMEM((3, CHUNK), jnp.int32),
          pltpu.VMEM((3, CHUNK), jnp.int32),
          pltpu.VMEM((CHUNK, d), jnp.float32),
          pltpu.VMEM((CHUNK, d), jnp.float32),
          pltpu.VMEM((zr, d), jnp.float32),
          pltpu.VMEM_SHARED((n_pad, d), jnp.float32),
          pltpu.SemaphoreType.DMA,
          pltpu.SemaphoreType.DMA,
          pltpu.SemaphoreType.DMA,
          pltpu.SemaphoreType.DMA,
      ],
      compiler_params=pltpu.CompilerParams(
          use_tc_tiling_on_sc=False, needs_layout_passes=False),
  )


def _gcn_dense(s1, w1s, b1s, w2s, bm=1000):
  nb, n, f = s1.shape
  h2 = w1s.shape[2]
  h = w2s.shape[2]

  def body(s1_ref, w1_ref, b1_ref, w2_ref, y_ref):
    hh = jnp.maximum(
        jnp.dot(s1_ref[0], w1_ref[0], preferred_element_type=jnp.float32)
        + b1_ref[0, 0], 0.0)
    y_ref[0] = jnp.dot(hh, w2_ref[0], preferred_element_type=jnp.float32)

  return pl.pallas_call(
      body,
      grid=(nb, n // bm),
      in_specs=[
          pl.BlockSpec((1, bm, f), lambda b, i: (b, i, 0)),
          pl.BlockSpec((1, f, h2), lambda b, i: (b, 0, 0)),
          pl.BlockSpec((1, 1, h2), lambda b, i: (b, 0, 0)),
          pl.BlockSpec((1, h2, h), lambda b, i: (b, 0, 0)),
      ],
      out_specs=pl.BlockSpec((1, bm, h), lambda b, i: (b, i, 0)),
      out_shape=jax.ShapeDtypeStruct((nb, n, h), jnp.float32),
  )(s1, w1s, b1s.reshape(nb, 1, h2), w2s)


def _lstm_diff(g, b2s, wih_t, whh_t, b_ih, b_hh, h_prev, c_prev, bm=1000):
  _, n, hdim = g.shape

  def body(g_ref, b2_ref, wih_ref, whh_ref, bih_ref, bhh_ref, hp_ref, cp_ref,
           d_ref):
    bias = bih_ref[...] + bhh_ref[...]

    def branch(b):
      gb = g_ref[b] + b2_ref[b, 0]
      gates = (jnp.dot(gb, wih_ref[...], preferred_element_type=jnp.float32)
               + jnp.dot(hp_ref[b], whh_ref[...],
                         preferred_element_type=jnp.float32)
               + bias)
      i_g = jax.nn.sigmoid(gates[:, :hdim])
      f_g = jax.nn.sigmoid(gates[:, hdim:2 * hdim])
      g_g = jnp.tanh(gates[:, 2 * hdim:3 * hdim])
      o_g = jax.nn.sigmoid(gates[:, 3 * hdim:])
      c_new = f_g * cp_ref[b] + i_g * g_g
      return o_g * jnp.tanh(c_new)

    d_ref[...] = branch(0) - branch(1)

  return pl.pallas_call(
      body,
      grid=(n // bm,),
      in_specs=[
          pl.BlockSpec((2, bm, hdim), lambda i: (0, i, 0)),
          pl.BlockSpec((2, 1, hdim), lambda i: (0, 0, 0)),
          pl.BlockSpec((hdim, 4 * hdim), lambda i: (0, 0)),
          pl.BlockSpec((hdim, 4 * hdim), lambda i: (0, 0)),
          pl.BlockSpec((4 * hdim,), lambda i: (0,)),
          pl.BlockSpec((4 * hdim,), lambda i: (0,)),
          pl.BlockSpec((2, bm, hdim), lambda i: (0, i, 0)),
          pl.BlockSpec((2, bm, hdim), lambda i: (0, i, 0)),
      ],
      out_specs=pl.BlockSpec((bm, hdim), lambda i: (i, 0)),
      out_shape=jax.ShapeDtypeStruct((n, hdim), jnp.float32),
  )(g, b2s.reshape(2, 1, hdim), wih_t, whh_t, b_ih, b_hh, h_prev, c_prev)


def _head(adj, d, feat, hid, wf, bf, w1, b1, w2, b2, bm=400):
  n = adj.shape[0]
  hdim = d.shape[1]
  f = feat.shape[1]
  o = w2.shape[1]

  def body(adj_ref, d_ref, feat_ref, hid_ref, wf_ref, bf_ref, w1_ref, b1_ref,
           w2_ref, b2_ref, ne_ref, y_ref):
    acc = jnp.dot(adj_ref[...], d_ref[...], preferred_element_type=jnp.float32)
    ne = (hid_ref[...] + bf_ref[...]
          + jnp.dot(feat_ref[...], wf_ref[...],
                    preferred_element_type=jnp.float32)
          + acc)
    ne_ref[...] = ne
    t = jnp.maximum(
        jnp.dot(ne, w1_ref[...], preferred_element_type=jnp.float32)
        + b1_ref[...], 0.0)
    y_ref[...] = (jnp.dot(t, w2_ref[...], preferred_element_type=jnp.float32)
                  + b2_ref[...])

  return pl.pallas_call(
      body,
      grid=(n // bm,),
      in_specs=[
          pl.BlockSpec((bm, n), lambda i: (i, 0)),
          pl.BlockSpec((n, hdim), lambda i: (0, 0)),
          pl.BlockSpec((bm, f), lambda i: (i, 0)),
          pl.BlockSpec((bm, hdim), lambda i: (i, 0)),
          pl.BlockSpec((f, hdim), lambda i: (0, 0)),
          pl.BlockSpec((hdim,), lambda i: (0,)),
          pl.BlockSpec((hdim, hdim), lambda i: (0, 0)),
          pl.BlockSpec((hdim,), lambda i: (0,)),
          pl.BlockSpec((hdim, o), lambda i: (0, 0)),
          pl.BlockSpec((o,), lambda i: (0,)),
      ],
      out_specs=[
          pl.BlockSpec((bm, hdim), lambda i: (i, 0)),
          pl.BlockSpec((bm, o), lambda i: (i, 0)),
      ],
      out_shape=[
          jax.ShapeDtypeStruct((n, hdim), jnp.float32),
          jax.ShapeDtypeStruct((n, o), jnp.float32),
      ],
  )(adj, d, feat, hid, wf, bf, w1, b1, w2, b2)


def kernel(features, edge_index_v, edge_vals_v, edge_index_p, edge_vals_p,
           adj_d, last_hidden_v, last_c_v, last_hidden_p, last_c_p,
           hidden_embedding, params):
  p = params
  n, f = features.shape
  e = edge_vals_v.shape[0]
  fp = 32

  feat32 = jnp.pad(features, ((0, 0), (0, fp - f)))

  t = -(-e // (NS * CHUNK))
  pad_len = t * NS * CHUNK - e

  def pack(r, co, va):
    rp = jnp.concatenate([r, jnp.zeros((pad_len,), jnp.int32)])
    cp = jnp.concatenate([co, jnp.zeros((pad_len,), jnp.int32)])
    vp = jnp.concatenate([va, jnp.zeros((pad_len,), jnp.float32)])
    return jnp.stack([
        rp.reshape(-1, CHUNK), cp.reshape(-1, CHUNK),
        lax.bitcast_convert_type(vp, jnp.int32).reshape(-1, CHUNK)], axis=1)

  packed = jnp.concatenate([
      pack(edge_index_v[0], edge_index_v[1], edge_vals_v),
      pack(edge_index_p[0], edge_index_p[1], edge_vals_p)])

  w1s = jnp.stack([
      jnp.pad(p['gcn_v_W1'], ((0, fp - f), (0, 0))),
      jnp.pad(p['gcn_p_W1'], ((0, fp - f), (0, 0))),
  ])
  b1s = jnp.stack([p['gcn_v_b1'], p['gcn_p_b1']])
  w2s = jnp.stack([p['gcn_v_W2'], p['gcn_p_W2']])
  b2s = jnp.stack([p['gcn_v_b2'], p['gcn_p_b2']])

  s1 = _make_spmm(n, t, fp, offset_cols=False)(feat32, packed)[:, :n, :]
  y = _gcn_dense(s1, w1s, b1s, w2s)
  g = _make_spmm(n, t, y.shape[2], offset_cols=True)(
      y.reshape(2 * n, y.shape[2]), packed)[:, :n, :]

  h_prev = jnp.stack([last_hidden_v, last_hidden_p])
  c_prev = jnp.stack([last_c_v, last_c_p])
  d = _lstm_diff(g, b2s, p['lstm_W_ih'].T, p['lstm_W_hh'].T,
                 p['lstm_b_ih'], p['lstm_b_hh'], h_prev, c_prev)

  wf = jnp.pad(p['lin_Wf'], ((0, fp - f), (0, 0)))
  node_embedding, y_pred = _head(
      adj_d, d, feat32, hidden_embedding, wf, p['lin_bf'],
      p['mlp_W1'], p['mlp_b1'], p['mlp_W2'], p['mlp_b2'])
  return node_embedding, y_pred
